# Initial kernel scaffold; baseline (speedup 1.0000x reference)
import jax
import jax.numpy as jnp
from jax.experimental import pallas as pl


def kernel(x, edge_index, W1, a_src1, a_dst1, b1, W2, a_src2, a_dst2, b2, Wm1, bm1, g1, be1, Wm2, bm2, g2, be2, Wm3, bm3, g3, be3, Wout, bout):
    raise NotImplementedError("write your pallas kernel here")


# speedup vs baseline: 1.0700x; 1.0700x over previous
import jax
import jax.numpy as jnp
from jax.experimental import pallas as pl
from jax.experimental import pallas as _pl

N = 10000
E = 320000
H = 8
C = 8

_SQ = float((1.0 + 1e-5) ** -0.5)


def _gat_nomax(x, src, dst, W, a_src, a_dst, b):
    n = x.shape[0]
    h = (x @ W).reshape(n, H, C)
    asrc = (h * a_src).sum(-1)
    adst = (h * a_dst).sum(-1)
    e = jax.nn.leaky_relu(asrc[src] + adst[dst], 0.2)
    ex = jnp.exp(e)
    s = jax.ops.segment_sum(ex, dst, num_segments=n)
    num = jax.ops.segment_sum(h[src] * ex[:, :, None], dst, num_segments=n)
    out = num / (s[:, :, None] + 1e-16)
    return out.reshape(n, H * C) + b


def _head_block(h_ref, wm1, bm1, g1, be1, wm2, bm2, g2, be2, wm3, bm3, g3, be3,
                wout, bout, o_ref):
    z = jnp.maximum(jnp.dot(h_ref[...], wm1[...]) + bm1[...], 0.0)
    z = g1[...] * z * _SQ + be1[...]
    z = jnp.maximum(jnp.dot(z, wm2[...]) + bm2[...], 0.0)
    z = g2[...] * z * _SQ + be2[...]
    z = jnp.maximum(jnp.dot(z, wm3[...]) + bm3[...], 0.0)
    z = g3[...] * z * _SQ + be3[...]
    o_ref[...] = jnp.dot(z, wout[...]) + bout[...]


def _mlp_head(h, Wm1, bm1, g1, be1, Wm2, bm2, g2, be2, Wm3, bm3, g3, be3,
              Wout, bout):
    n = h.shape[0]
    npad = 10240
    hp = jnp.pad(h, ((0, npad - n), (0, 0)))
    blk = 1024
    grid = (npad // blk,)
    full = lambda s: pl.BlockSpec(s, lambda i: tuple(0 for _ in s))
    out = pl.pallas_call(
        _head_block,
        grid=grid,
        in_specs=[pl.BlockSpec((blk, 64), lambda i: (i, 0)),
                  full((64, 32)), full((32,)), full((32,)), full((32,)),
                  full((32, 16)), full((16,)), full((16,)), full((16,)),
                  full((16, 8)), full((8,)), full((8,)), full((8,)),
                  full((8, 8)), full((8,))],
        out_specs=pl.BlockSpec((blk, 8), lambda i: (i, 0)),
        out_shape=jax.ShapeDtypeStruct((npad, 8), jnp.float32),
    )(hp, Wm1, bm1, g1, be1, Wm2, bm2, g2, be2, Wm3, bm3, g3, be3, Wout, bout)
    return out[:n]


def kernel(x, edge_index, W1, a_src1, a_dst1, b1, W2, a_src2, a_dst2, b2,
           Wm1, bm1, g1, be1, Wm2, bm2, g2, be2, Wm3, bm3, g3, be3,
           Wout, bout):
    n = x.shape[0]
    loop = jnp.arange(n, dtype=edge_index.dtype)
    src = jnp.concatenate([edge_index[0], loop])
    dst = jnp.concatenate([edge_index[1], loop])
    h = jax.nn.elu(_gat_nomax(x, src, dst, W1, a_src1, a_dst1, b1))
    h = jax.nn.elu(_gat_nomax(h, src, dst, W2, a_src2, a_dst2, b2))
    return _mlp_head(h, Wm1, bm1, g1, be1, Wm2, bm2, g2, be2, Wm3, bm3,
                     g3, be3, Wout, bout)


# speedup vs baseline: 62.7028x; 58.6024x over previous
import dataclasses
import functools

import jax
import jax.numpy as jnp
from jax import lax
from jax.experimental import pallas as pl
from jax.experimental.pallas import tpu as pltpu
from jax.experimental.pallas import tpu_sc as plsc

N = 10000
H = 8
C = 8
F = 64

NP = 10240
GW = 80
DW = 16
BLK = 1280
NTC = NP // BLK

NCORE = 2
NSUB = 16
NWORK = NCORE * NSUB
K = 128
_SQ = float((1.0 + 1e-5) ** -0.5)


def _sel(shape, fn):
    i0 = lax.broadcasted_iota(jnp.int32, shape, 0)
    i1 = lax.broadcasted_iota(jnp.int32, shape, 1)
    return fn(i0, i1).astype(jnp.float32)


def _tables_block(x_ref, w_ref, asrc_ref, adst_ref, g_ref, d_ref):
    h = jnp.dot(x_ref[...], w_ref[...], preferred_element_type=jnp.float32)
    p1 = _sel((F, GW), lambda i, j: i == j)
    s80 = _sel((F, GW), lambda i, j: j == F + i // C)
    s16 = _sel((F, DW), lambda i, j: j == i // C)
    g_ref[...] = jnp.dot(h, p1, preferred_element_type=jnp.float32) + jnp.dot(
        h * asrc_ref[...], s80, preferred_element_type=jnp.float32)
    d_ref[...] = jnp.dot(h * adst_ref[...], s16,
                         preferred_element_type=jnp.float32)


def _tables(x, W, asrc_vec, adst_vec):
    fin = x.shape[1]
    full = lambda s: pl.BlockSpec(s, lambda i: tuple(0 for _ in s))
    return pl.pallas_call(
        _tables_block,
        grid=(NTC,),
        in_specs=[pl.BlockSpec((BLK, fin), lambda i: (i, 0)),
                  full((fin, F)), full((1, F)), full((1, F))],
        out_specs=[pl.BlockSpec((BLK, GW), lambda i: (i, 0)),
                   pl.BlockSpec((BLK, DW), lambda i: (i, 0))],
        out_shape=[jax.ShapeDtypeStruct((NP, GW), jnp.float32),
                   jax.ShapeDtypeStruct((NP, DW), jnp.float32)],
    )(x, W, asrc_vec, adst_vec)


def _combine_block(a0_ref, a1_ref, b_ref, o_ref):
    t = a0_ref[...] + a1_ref[...]
    nmat = _sel((GW, F), lambda i, j: i == j)
    bmat = _sel((GW, F), lambda i, j: i == F + j // C)
    num = jnp.dot(t, nmat, preferred_element_type=jnp.float32)
    den = jnp.dot(t, bmat, preferred_element_type=jnp.float32)
    out = num / (den + 1e-16) + b_ref[...]
    o_ref[...] = jnp.where(out > 0.0, out, jnp.exp(out) - 1.0)


def _combine_elu(acc0, acc1, b):
    full = lambda s: pl.BlockSpec(s, lambda i: tuple(0 for _ in s))
    return pl.pallas_call(
        _combine_block,
        grid=(NTC,),
        in_specs=[pl.BlockSpec((BLK, GW), lambda i: (i, 0)),
                  pl.BlockSpec((BLK, GW), lambda i: (i, 0)),
                  full((1, F))],
        out_specs=pl.BlockSpec((BLK, F), lambda i: (i, 0)),
        out_shape=jax.ShapeDtypeStruct((NP, F), jnp.float32),
    )(acc0, acc1, b)


def _head_block(h_ref, wm1, bm1, g1, be1, wm2, bm2, g2, be2, wm3, bm3, g3,
                be3, wout, bout, o_ref):
    z = jnp.maximum(jnp.dot(h_ref[...], wm1[...],
                            preferred_element_type=jnp.float32) + bm1[...], 0.0)
    z = g1[...] * z * _SQ + be1[...]
    z = jnp.maximum(jnp.dot(z, wm2[...],
                            preferred_element_type=jnp.float32) + bm2[...], 0.0)
    z = g2[...] * z * _SQ + be2[...]
    z = jnp.maximum(jnp.dot(z, wm3[...],
                            preferred_element_type=jnp.float32) + bm3[...], 0.0)
    z = g3[...] * z * _SQ + be3[...]
    o_ref[...] = jnp.dot(z, wout[...],
                         preferred_element_type=jnp.float32) + bout[...]


def _mlp_head(h, Wm1, bm1, g1, be1, Wm2, bm2, g2, be2, Wm3, bm3, g3, be3,
              Wout, bout):
    full = lambda s: pl.BlockSpec(s, lambda i: tuple(0 for _ in s))
    return pl.pallas_call(
        _head_block,
        grid=(NTC,),
        in_specs=[pl.BlockSpec((BLK, F), lambda i: (i, 0)),
                  full((F, 32)), full((1, 32)), full((1, 32)), full((1, 32)),
                  full((32, 16)), full((1, 16)), full((1, 16)), full((1, 16)),
                  full((16, 8)), full((1, 8)), full((1, 8)), full((1, 8)),
                  full((8, 8)), full((1, 8))],
        out_specs=pl.BlockSpec((BLK, 8), lambda i: (i, 0)),
        out_shape=jax.ShapeDtypeStruct((NP, 8), jnp.float32),
    )(h, Wm1, bm1, g1, be1, Wm2, bm2, g2, be2, Wm3, bm3, g3, be3, Wout, bout)


def _edge_pass_body(g_hbm, d_hbm, src_hbm, dst_hbm, zero_hbm, out_hbm,
                    srcv, dstv, gbuf, dbuf, sbuf, acc):
    cid = lax.axis_index("c")
    sid = lax.axis_index("s")
    nb = src_hbm.shape[1]
    rows_per_tile = NP // NSUB

    pltpu.sync_copy(zero_hbm, acc.at[pl.ds(sid * rows_per_tile, rows_per_tile)])
    plsc.subcore_barrier()

    wid = sid * NCORE + cid
    pltpu.sync_copy(src_hbm.at[wid], srcv)
    pltpu.sync_copy(dst_hbm.at[wid], dstv)

    lane = lax.iota(jnp.int32, 16)
    col_hi = lane >> 3
    bcast_cols = [F + 2 * j + col_hi for j in range(4)]

    @pl.loop(0, nb)
    def _blk(b):
        pltpu.sync_copy(g_hbm.at[srcv.at[b]], gbuf)
        pltpu.sync_copy(d_hbm.at[dstv.at[b]], dbuf)

        @pl.loop(0, K)
        def _edge(i):
            a = gbuf[i, pl.ds(F, 16)]
            d = dbuf[i, pl.ds(0, 16)]
            e = a + d
            e = jnp.maximum(e, e * 0.2)
            ex = jnp.exp(e)
            sbuf[i, pl.ds(F, 16)] = ex
            rowv = jnp.full((16,), i, dtype=jnp.int32)
            for j in range(4):
                bc = plsc.load_gather(sbuf, [rowv, bcast_cols[j]])
                sbuf[i, pl.ds(16 * j, 16)] = gbuf[i, pl.ds(16 * j, 16)] * bc

        pltpu.sync_copy(sbuf, acc.at[dstv.at[b]], add=True)

    plsc.subcore_barrier()
    pltpu.sync_copy(acc.at[pl.ds(sid * rows_per_tile, rows_per_tile)],
                    out_hbm.at[cid].at[pl.ds(sid * rows_per_tile,
                                             rows_per_tile)])


def _edge_pass(g, d, src3, dst3, zero_rows):
    mesh = plsc.VectorSubcoreMesh(core_axis_name="c", subcore_axis_name="s")
    nb = src3.shape[1]
    cp = pltpu.CompilerParams(needs_layout_passes=False,
                              use_tc_tiling_on_sc=False)
    kern = functools.partial(
        pl.kernel,
        compiler_params=cp,
        out_type=jax.ShapeDtypeStruct((NCORE, NP, GW), jnp.float32),
        mesh=mesh,
        scratch_types=[
            pltpu.VMEM((nb, K), jnp.int32),
            pltpu.VMEM((nb, K), jnp.int32),
            pltpu.VMEM((K, GW), jnp.float32),
            pltpu.VMEM((K, DW), jnp.float32),
            pltpu.VMEM((K, GW), jnp.float32),
            pltpu.VMEM_SHARED((NP, GW), jnp.float32),
        ],
    )(_edge_pass_body)
    return kern(g, d, src3, dst3, zero_rows)


def kernel(x, edge_index, W1, a_src1, a_dst1, b1, W2, a_src2, a_dst2, b2,
           Wm1, bm1, g1, be1, Wm2, bm2, g2, be2, Wm3, bm3, g3, be3,
           Wout, bout):
    n = x.shape[0]
    loop = jnp.arange(n, dtype=edge_index.dtype)
    src = jnp.concatenate([edge_index[0], loop])
    dst = jnp.concatenate([edge_index[1], loop])
    e_tot = src.shape[0]
    ept = ((e_tot + NWORK * K - 1) // (NWORK * K)) * K
    epad = NWORK * ept - e_tot
    trash = jnp.full((epad,), N, dtype=src.dtype)
    src3 = jnp.concatenate([src, trash]).reshape(NWORK, ept // K, K)
    dst3 = jnp.concatenate([dst, trash]).reshape(NWORK, ept // K, K)

    xp = jnp.pad(x, ((0, NP - n), (0, 0)))
    zero_rows = jnp.zeros((NP // NSUB, GW), jnp.float32)

    g1t, d1t = _tables(xp, W1, a_src1.reshape(1, F), a_dst1.reshape(1, F))
    acc1 = _edge_pass(g1t, d1t, src3, dst3, zero_rows)
    h1 = _combine_elu(acc1[0], acc1[1], b1.reshape(1, F))

    g2t, d2t = _tables(h1, W2, a_src2.reshape(1, F), a_dst2.reshape(1, F))
    acc2 = _edge_pass(g2t, d2t, src3, dst3, zero_rows)
    h2 = _combine_elu(acc2[0], acc2[1], b2.reshape(1, F))

    out = _mlp_head(h2, Wm1, bm1.reshape(1, 32), g1.reshape(1, 32),
                    be1.reshape(1, 32), Wm2, bm2.reshape(1, 16),
                    g2.reshape(1, 16), be2.reshape(1, 16), Wm3,
                    bm3.reshape(1, 8), g3.reshape(1, 8), be3.reshape(1, 8),
                    Wout, bout.reshape(1, 8))
    return out[:n]


# speedup vs baseline: 65.8352x; 1.0500x over previous
import dataclasses
import functools

import jax
import jax.numpy as jnp
from jax import lax
from jax.experimental import pallas as pl
from jax.experimental.pallas import tpu as pltpu
from jax.experimental.pallas import tpu_sc as plsc

N = 10000
H = 8
C = 8
F = 64

NP = 10240
GW = 80
DW = 16
BLK = 1280
NTC = NP // BLK

NCORE = 2
NSUB = 16
NWORK = NCORE * NSUB
K = 128
_SQ = float((1.0 + 1e-5) ** -0.5)


def _sel(shape, fn):
    i0 = lax.broadcasted_iota(jnp.int32, shape, 0)
    i1 = lax.broadcasted_iota(jnp.int32, shape, 1)
    return fn(i0, i1).astype(jnp.float32)


def _tables_block(x_ref, w_ref, asrc_ref, adst_ref, g_ref, d_ref):
    h = jnp.dot(x_ref[...], w_ref[...], preferred_element_type=jnp.float32, precision=lax.Precision.HIGHEST)
    p1 = _sel((F, GW), lambda i, j: i == j)
    s80 = _sel((F, GW), lambda i, j: j == F + i // C)
    s16 = _sel((F, DW), lambda i, j: j == i // C)
    g_ref[...] = jnp.dot(h, p1, preferred_element_type=jnp.float32, precision=lax.Precision.HIGHEST) + jnp.dot(
        h * asrc_ref[...], s80, preferred_element_type=jnp.float32, precision=lax.Precision.HIGHEST)
    d_ref[...] = jnp.dot(h * adst_ref[...], s16,
                         preferred_element_type=jnp.float32, precision=lax.Precision.HIGHEST)


def _tables(x, W, asrc_vec, adst_vec):
    fin = x.shape[1]
    full = lambda s: pl.BlockSpec(s, lambda i: tuple(0 for _ in s))
    return pl.pallas_call(
        _tables_block,
        grid=(NTC,),
        in_specs=[pl.BlockSpec((BLK, fin), lambda i: (i, 0)),
                  full((fin, F)), full((1, F)), full((1, F))],
        out_specs=[pl.BlockSpec((BLK, GW), lambda i: (i, 0)),
                   pl.BlockSpec((BLK, DW), lambda i: (i, 0))],
        out_shape=[jax.ShapeDtypeStruct((NP, GW), jnp.float32),
                   jax.ShapeDtypeStruct((NP, DW), jnp.float32)],
    )(x, W, asrc_vec, adst_vec)


def _combine_block(a0_ref, a1_ref, b_ref, o_ref):
    t = a0_ref[...] + a1_ref[...]
    nmat = _sel((GW, F), lambda i, j: i == j)
    bmat = _sel((GW, F), lambda i, j: i == F + j // C)
    num = jnp.dot(t, nmat, preferred_element_type=jnp.float32, precision=lax.Precision.HIGHEST)
    den = jnp.dot(t, bmat, preferred_element_type=jnp.float32, precision=lax.Precision.HIGHEST)
    out = num / (den + 1e-16) + b_ref[...]
    o_ref[...] = jnp.where(out > 0.0, out, jnp.exp(out) - 1.0)


def _combine_elu(acc0, acc1, b):
    full = lambda s: pl.BlockSpec(s, lambda i: tuple(0 for _ in s))
    return pl.pallas_call(
        _combine_block,
        grid=(NTC,),
        in_specs=[pl.BlockSpec((BLK, GW), lambda i: (i, 0)),
                  pl.BlockSpec((BLK, GW), lambda i: (i, 0)),
                  full((1, F))],
        out_specs=pl.BlockSpec((BLK, F), lambda i: (i, 0)),
        out_shape=jax.ShapeDtypeStruct((NP, F), jnp.float32),
    )(acc0, acc1, b)


def _head_block(h_ref, wm1, bm1, g1, be1, wm2, bm2, g2, be2, wm3, bm3, g3,
                be3, wout, bout, o_ref):
    z = jnp.maximum(jnp.dot(h_ref[...], wm1[...],
                            preferred_element_type=jnp.float32, precision=lax.Precision.HIGHEST) + bm1[...], 0.0)
    z = g1[...] * z * _SQ + be1[...]
    z = jnp.maximum(jnp.dot(z, wm2[...],
                            preferred_element_type=jnp.float32, precision=lax.Precision.HIGHEST) + bm2[...], 0.0)
    z = g2[...] * z * _SQ + be2[...]
    z = jnp.maximum(jnp.dot(z, wm3[...],
                            preferred_element_type=jnp.float32, precision=lax.Precision.HIGHEST) + bm3[...], 0.0)
    z = g3[...] * z * _SQ + be3[...]
    o_ref[...] = jnp.dot(z, wout[...],
                         preferred_element_type=jnp.float32, precision=lax.Precision.HIGHEST) + bout[...]


def _mlp_head(h, Wm1, bm1, g1, be1, Wm2, bm2, g2, be2, Wm3, bm3, g3, be3,
              Wout, bout):
    full = lambda s: pl.BlockSpec(s, lambda i: tuple(0 for _ in s))
    return pl.pallas_call(
        _head_block,
        grid=(NTC,),
        in_specs=[pl.BlockSpec((BLK, F), lambda i: (i, 0)),
                  full((F, 32)), full((1, 32)), full((1, 32)), full((1, 32)),
                  full((32, 16)), full((1, 16)), full((1, 16)), full((1, 16)),
                  full((16, 8)), full((1, 8)), full((1, 8)), full((1, 8)),
                  full((8, 8)), full((1, 8))],
        out_specs=pl.BlockSpec((BLK, 8), lambda i: (i, 0)),
        out_shape=jax.ShapeDtypeStruct((NP, 8), jnp.float32),
    )(h, Wm1, bm1, g1, be1, Wm2, bm2, g2, be2, Wm3, bm3, g3, be3, Wout, bout)


def _edge_pass_body(g_hbm, d_hbm, src_hbm, dst_hbm, zero_hbm, out_hbm,
                    srcv, dstv, gbuf0, gbuf1, dbuf0, dbuf1, sbuf0, sbuf1,
                    acc, gsem0, gsem1, dsem0, dsem1):
    cid = lax.axis_index("c")
    sid = lax.axis_index("s")
    nb = src_hbm.shape[1]
    rpt = NP // NSUB

    pltpu.sync_copy(zero_hbm, acc.at[pl.ds(sid * rpt, rpt)])

    wid = sid * NCORE + cid
    pltpu.sync_copy(src_hbm.at[wid], srcv)
    pltpu.sync_copy(dst_hbm.at[wid], dstv)

    lane = lax.iota(jnp.int32, 16)
    hi = lane >> 3
    lo = lane & 7
    col_a = F + lo
    zero16 = jnp.zeros((16,), jnp.float32)
    bcast_cols = [F + 2 * j + hi for j in range(4)]

    @pl.loop(0, K)
    def _init(i):
        sbuf0[i, pl.ds(F, 16)] = zero16

    plsc.subcore_barrier()

    def compute(gb, db, sb):
        @pl.loop(0, K, step=2)
        def _pair(i):
            rowp = hi + i
            a = plsc.load_gather(gb, [rowp, col_a])
            d = plsc.load_gather(db, [rowp, lo])
            e = a + d
            e = jnp.maximum(e, e * 0.2)
            ex = jnp.exp(e)
            plsc.store_scatter(sb, [rowp, col_a], ex)
            for k in range(2):
                row = jnp.full((16,), i + k, dtype=jnp.int32)
                for j in range(4):
                    bc = plsc.load_gather(sb, [row, bcast_cols[j]])
                    sb[i + k, pl.ds(16 * j, 16)] = (
                        gb[i + k, pl.ds(16 * j, 16)] * bc)

    @pl.loop(0, nb)
    def _blk(b):
        pltpu.sync_copy(g_hbm.at[srcv.at[b]], gbuf0)
        pltpu.sync_copy(d_hbm.at[dstv.at[b]], dbuf0)
        compute(gbuf0, dbuf0, sbuf0)
        pltpu.sync_copy(sbuf0, acc.at[dstv.at[b]], add=True)

    plsc.subcore_barrier()
    pltpu.sync_copy(acc.at[pl.ds(sid * rpt, rpt)],
                    out_hbm.at[cid].at[pl.ds(sid * rpt, rpt)])


def _edge_pass(g, d, src3, dst3, zero_rows):
    mesh = plsc.VectorSubcoreMesh(core_axis_name="c", subcore_axis_name="s")
    nb = src3.shape[1]
    cp = pltpu.CompilerParams(needs_layout_passes=False,
                              use_tc_tiling_on_sc=False)
    kern = functools.partial(
        pl.kernel,
        compiler_params=cp,
        out_type=jax.ShapeDtypeStruct((NCORE, NP, GW), jnp.float32),
        mesh=mesh,
        scratch_types=[
            pltpu.VMEM((nb, K), jnp.int32),
            pltpu.VMEM((nb, K), jnp.int32),
            pltpu.VMEM((K, GW), jnp.float32),
            pltpu.VMEM((K, GW), jnp.float32),
            pltpu.VMEM((K, DW), jnp.float32),
            pltpu.VMEM((K, DW), jnp.float32),
            pltpu.VMEM((K, GW), jnp.float32),
            pltpu.VMEM((K, GW), jnp.float32),
            pltpu.VMEM_SHARED((NP, GW), jnp.float32),
            pltpu.SemaphoreType.DMA,
            pltpu.SemaphoreType.DMA,
            pltpu.SemaphoreType.DMA,
            pltpu.SemaphoreType.DMA,
        ],
    )(_edge_pass_body)
    return kern(g, d, src3, dst3, zero_rows)


def kernel(x, edge_index, W1, a_src1, a_dst1, b1, W2, a_src2, a_dst2, b2,
           Wm1, bm1, g1, be1, Wm2, bm2, g2, be2, Wm3, bm3, g3, be3,
           Wout, bout):
    n = x.shape[0]
    loop = jnp.arange(n, dtype=edge_index.dtype)
    src = jnp.concatenate([edge_index[0], loop])
    dst = jnp.concatenate([edge_index[1], loop])
    e_tot = src.shape[0]
    ept = ((e_tot + NWORK * K - 1) // (NWORK * K)) * K
    epad = NWORK * ept - e_tot
    trash = jnp.full((epad,), N, dtype=src.dtype)
    src3 = jnp.concatenate([src, trash]).reshape(NWORK, ept // K, K)
    dst3 = jnp.concatenate([dst, trash]).reshape(NWORK, ept // K, K)

    xp = jnp.pad(x, ((0, NP - n), (0, 0)))
    zero_rows = jnp.zeros((NP // NSUB, GW), jnp.float32)

    g1t, d1t = _tables(xp, W1, a_src1.reshape(1, F), a_dst1.reshape(1, F))
    acc1 = _edge_pass(g1t, d1t, src3, dst3, zero_rows)
    h1 = _combine_elu(acc1[0], acc1[1], b1.reshape(1, F))

    g2t, d2t = _tables(h1, W2, a_src2.reshape(1, F), a_dst2.reshape(1, F))
    acc2 = _edge_pass(g2t, d2t, src3, dst3, zero_rows)
    h2 = _combine_elu(acc2[0], acc2[1], b2.reshape(1, F))

    out = _mlp_head(h2, Wm1, bm1.reshape(1, 32), g1.reshape(1, 32),
                    be1.reshape(1, 32), Wm2, bm2.reshape(1, 16),
                    g2.reshape(1, 16), be2.reshape(1, 16), Wm3,
                    bm3.reshape(1, 8), g3.reshape(1, 8), be3.reshape(1, 8),
                    Wout, bout.reshape(1, 8))
    return out[:n]


# speedup vs baseline: 72.2799x; 1.0979x over previous
import dataclasses
import functools

import jax
import jax.numpy as jnp
from jax import lax
from jax.experimental import pallas as pl
from jax.experimental.pallas import tpu as pltpu
from jax.experimental.pallas import tpu_sc as plsc

N = 10000
H = 8
C = 8
F = 64

NP = 10240
GW = 80
DW = 16
BLK = 1280
NTC = NP // BLK

NCORE = 2
NSUB = 16
NWORK = NCORE * NSUB
K = 128
_SQ = float((1.0 + 1e-5) ** -0.5)


def _sel(shape, fn):
    i0 = lax.broadcasted_iota(jnp.int32, shape, 0)
    i1 = lax.broadcasted_iota(jnp.int32, shape, 1)
    return fn(i0, i1).astype(jnp.float32)


def _tables_block(x_ref, w_ref, asrc_ref, adst_ref, g_ref, d_ref):
    h = jnp.dot(x_ref[...], w_ref[...], preferred_element_type=jnp.float32, precision=lax.Precision.HIGHEST)
    p1 = _sel((F, GW), lambda i, j: i == j)
    s80 = _sel((F, GW), lambda i, j: j == F + i // C)
    s16 = _sel((F, DW), lambda i, j: j == i // C)
    g_ref[...] = jnp.dot(h, p1, preferred_element_type=jnp.float32, precision=lax.Precision.HIGHEST) + jnp.dot(
        h * asrc_ref[...], s80, preferred_element_type=jnp.float32, precision=lax.Precision.HIGHEST)
    d_ref[...] = jnp.dot(h * adst_ref[...], s16,
                         preferred_element_type=jnp.float32, precision=lax.Precision.HIGHEST)


def _tables(x, W, asrc_vec, adst_vec):
    fin = x.shape[1]
    full = lambda s: pl.BlockSpec(s, lambda i: tuple(0 for _ in s))
    return pl.pallas_call(
        _tables_block,
        grid=(NTC,),
        in_specs=[pl.BlockSpec((BLK, fin), lambda i: (i, 0)),
                  full((fin, F)), full((1, F)), full((1, F))],
        out_specs=[pl.BlockSpec((BLK, GW), lambda i: (i, 0)),
                   pl.BlockSpec((BLK, DW), lambda i: (i, 0))],
        out_shape=[jax.ShapeDtypeStruct((NP, GW), jnp.float32),
                   jax.ShapeDtypeStruct((NP, DW), jnp.float32)],
    )(x, W, asrc_vec, adst_vec)


def _combine_block(a0_ref, a1_ref, b_ref, o_ref):
    t = a0_ref[...] + a1_ref[...]
    nmat = _sel((GW, F), lambda i, j: i == j)
    bmat = _sel((GW, F), lambda i, j: i == F + j // C)
    num = jnp.dot(t, nmat, preferred_element_type=jnp.float32, precision=lax.Precision.HIGHEST)
    den = jnp.dot(t, bmat, preferred_element_type=jnp.float32, precision=lax.Precision.HIGHEST)
    out = num / (den + 1e-16) + b_ref[...]
    o_ref[...] = jnp.where(out > 0.0, out, jnp.exp(out) - 1.0)


def _combine_elu(acc0, acc1, b):
    full = lambda s: pl.BlockSpec(s, lambda i: tuple(0 for _ in s))
    return pl.pallas_call(
        _combine_block,
        grid=(NTC,),
        in_specs=[pl.BlockSpec((BLK, GW), lambda i: (i, 0)),
                  pl.BlockSpec((BLK, GW), lambda i: (i, 0)),
                  full((1, F))],
        out_specs=pl.BlockSpec((BLK, F), lambda i: (i, 0)),
        out_shape=jax.ShapeDtypeStruct((NP, F), jnp.float32),
    )(acc0, acc1, b)


def _head_block(h_ref, wm1, bm1, g1, be1, wm2, bm2, g2, be2, wm3, bm3, g3,
                be3, wout, bout, o_ref):
    z = jnp.maximum(jnp.dot(h_ref[...], wm1[...],
                            preferred_element_type=jnp.float32, precision=lax.Precision.HIGHEST) + bm1[...], 0.0)
    z = g1[...] * z * _SQ + be1[...]
    z = jnp.maximum(jnp.dot(z, wm2[...],
                            preferred_element_type=jnp.float32, precision=lax.Precision.HIGHEST) + bm2[...], 0.0)
    z = g2[...] * z * _SQ + be2[...]
    z = jnp.maximum(jnp.dot(z, wm3[...],
                            preferred_element_type=jnp.float32, precision=lax.Precision.HIGHEST) + bm3[...], 0.0)
    z = g3[...] * z * _SQ + be3[...]
    o_ref[...] = jnp.dot(z, wout[...],
                         preferred_element_type=jnp.float32, precision=lax.Precision.HIGHEST) + bout[...]


def _mlp_head(h, Wm1, bm1, g1, be1, Wm2, bm2, g2, be2, Wm3, bm3, g3, be3,
              Wout, bout):
    full = lambda s: pl.BlockSpec(s, lambda i: tuple(0 for _ in s))
    return pl.pallas_call(
        _head_block,
        grid=(NTC,),
        in_specs=[pl.BlockSpec((BLK, F), lambda i: (i, 0)),
                  full((F, 32)), full((1, 32)), full((1, 32)), full((1, 32)),
                  full((32, 16)), full((1, 16)), full((1, 16)), full((1, 16)),
                  full((16, 8)), full((1, 8)), full((1, 8)), full((1, 8)),
                  full((8, 8)), full((1, 8))],
        out_specs=pl.BlockSpec((BLK, 8), lambda i: (i, 0)),
        out_shape=jax.ShapeDtypeStruct((NP, 8), jnp.float32),
    )(h, Wm1, bm1, g1, be1, Wm2, bm2, g2, be2, Wm3, bm3, g3, be3, Wout, bout)


def _edge_pass_body(g_hbm, d_hbm, src_hbm, dst_hbm, zero_hbm, out_hbm,
                    srcv, dstv, gbuf0, gbuf1, dbuf0, dbuf1, sbuf0, sbuf1,
                    acc, gsem0, gsem1, dsem0, dsem1):
    cid = lax.axis_index("c")
    sid = lax.axis_index("s")
    nb = src_hbm.shape[1]
    rpt = NP // NSUB

    pltpu.sync_copy(zero_hbm, acc.at[pl.ds(sid * rpt, rpt)])

    wid = sid * NCORE + cid
    pltpu.sync_copy(src_hbm.at[wid], srcv)
    pltpu.sync_copy(dst_hbm.at[wid], dstv)

    lane = lax.iota(jnp.int32, 16)
    hi = lane >> 3
    lo = lane & 7
    col_a = F + lo
    zero16 = jnp.zeros((16,), jnp.float32)
    bcast_cols = [F + 2 * j + hi for j in range(4)]

    @pl.loop(0, K)
    def _init(i):
        sbuf0[i, pl.ds(F, 16)] = zero16

    plsc.subcore_barrier()

    def compute(gb, db, sb):
        @pl.loop(0, K, step=2)
        def _pair(i):
            rowp = hi + i
            a = plsc.load_gather(gb, [rowp, col_a])
            d = plsc.load_gather(db, [rowp, lo])
            e = a + d
            e = jnp.maximum(e, e * 0.2)
            ex = jnp.exp(e)
            plsc.store_scatter(sb, [rowp, col_a], ex)
            for k in range(2):
                row = jnp.full((16,), i + k, dtype=jnp.int32)
                for j in range(4):
                    bc = plsc.load_gather(sb, [row, bcast_cols[j]])
                    sb[i + k, pl.ds(16 * j, 16)] = (
                        gb[i + k, pl.ds(16 * j, 16)] * bc)

    @pl.loop(0, nb)
    def _blk(b):
        cg = pltpu.async_copy(g_hbm.at[srcv.at[b]], gbuf0, gsem0)
        cd = pltpu.async_copy(d_hbm.at[dstv.at[b]], dbuf0, dsem0)
        cg.wait()
        cd.wait()
        compute(gbuf0, dbuf0, sbuf0)
        pltpu.sync_copy(sbuf0, acc.at[dstv.at[b]], add=True)

    plsc.subcore_barrier()
    pltpu.sync_copy(acc.at[pl.ds(sid * rpt, rpt)],
                    out_hbm.at[cid].at[pl.ds(sid * rpt, rpt)])


def _edge_pass(g, d, src3, dst3, zero_rows):
    mesh = plsc.VectorSubcoreMesh(core_axis_name="c", subcore_axis_name="s")
    nb = src3.shape[1]
    cp = pltpu.CompilerParams(needs_layout_passes=False,
                              use_tc_tiling_on_sc=False)
    kern = functools.partial(
        pl.kernel,
        compiler_params=cp,
        out_type=jax.ShapeDtypeStruct((NCORE, NP, GW), jnp.float32),
        mesh=mesh,
        scratch_types=[
            pltpu.VMEM((nb, K), jnp.int32),
            pltpu.VMEM((nb, K), jnp.int32),
            pltpu.VMEM((K, GW), jnp.float32),
            pltpu.VMEM((K, GW), jnp.float32),
            pltpu.VMEM((K, DW), jnp.float32),
            pltpu.VMEM((K, DW), jnp.float32),
            pltpu.VMEM((K, GW), jnp.float32),
            pltpu.VMEM((K, GW), jnp.float32),
            pltpu.VMEM_SHARED((NP, GW), jnp.float32),
            pltpu.SemaphoreType.DMA,
            pltpu.SemaphoreType.DMA,
            pltpu.SemaphoreType.DMA,
            pltpu.SemaphoreType.DMA,
        ],
    )(_edge_pass_body)
    return kern(g, d, src3, dst3, zero_rows)


def kernel(x, edge_index, W1, a_src1, a_dst1, b1, W2, a_src2, a_dst2, b2,
           Wm1, bm1, g1, be1, Wm2, bm2, g2, be2, Wm3, bm3, g3, be3,
           Wout, bout):
    n = x.shape[0]
    loop = jnp.arange(n, dtype=edge_index.dtype)
    src = jnp.concatenate([edge_index[0], loop])
    dst = jnp.concatenate([edge_index[1], loop])
    e_tot = src.shape[0]
    ept = ((e_tot + NWORK * K - 1) // (NWORK * K)) * K
    epad = NWORK * ept - e_tot
    trash = jnp.full((epad,), N, dtype=src.dtype)
    src3 = jnp.concatenate([src, trash]).reshape(NWORK, ept // K, K)
    dst3 = jnp.concatenate([dst, trash]).reshape(NWORK, ept // K, K)

    xp = jnp.pad(x, ((0, NP - n), (0, 0)))
    zero_rows = jnp.zeros((NP // NSUB, GW), jnp.float32)

    g1t, d1t = _tables(xp, W1, a_src1.reshape(1, F), a_dst1.reshape(1, F))
    acc1 = _edge_pass(g1t, d1t, src3, dst3, zero_rows)
    h1 = _combine_elu(acc1[0], acc1[1], b1.reshape(1, F))

    g2t, d2t = _tables(h1, W2, a_src2.reshape(1, F), a_dst2.reshape(1, F))
    acc2 = _edge_pass(g2t, d2t, src3, dst3, zero_rows)
    h2 = _combine_elu(acc2[0], acc2[1], b2.reshape(1, F))

    out = _mlp_head(h2, Wm1, bm1.reshape(1, 32), g1.reshape(1, 32),
                    be1.reshape(1, 32), Wm2, bm2.reshape(1, 16),
                    g2.reshape(1, 16), be2.reshape(1, 16), Wm3,
                    bm3.reshape(1, 8), g3.reshape(1, 8), be3.reshape(1, 8),
                    Wout, bout.reshape(1, 8))
    return out[:n]


# speedup vs baseline: 72.2954x; 1.0002x over previous
import dataclasses
import functools

import jax
import jax.numpy as jnp
from jax import lax
from jax.experimental import pallas as pl
from jax.experimental.pallas import tpu as pltpu
from jax.experimental.pallas import tpu_sc as plsc

N = 10000
H = 8
C = 8
F = 64

NP = 10240
GW = 80
DW = 16
BLK = 1280
NTC = NP // BLK

NCORE = 2
NSUB = 16
NWORK = NCORE * NSUB
K = 128
_SQ = float((1.0 + 1e-5) ** -0.5)


def _sel(shape, fn):
    i0 = lax.broadcasted_iota(jnp.int32, shape, 0)
    i1 = lax.broadcasted_iota(jnp.int32, shape, 1)
    return fn(i0, i1).astype(jnp.float32)


def _tables_block(x_ref, w_ref, asrc_ref, adst_ref, g_ref, d_ref):
    h = jnp.dot(x_ref[...], w_ref[...], preferred_element_type=jnp.float32, precision=lax.Precision.HIGHEST)
    p1 = _sel((F, GW), lambda i, j: i == j)
    s80 = _sel((F, GW), lambda i, j: j == F + i // C)
    s16 = _sel((F, DW), lambda i, j: j == i // C)
    g_ref[...] = jnp.dot(h, p1, preferred_element_type=jnp.float32, precision=lax.Precision.HIGHEST) + jnp.dot(
        h * asrc_ref[...], s80, preferred_element_type=jnp.float32, precision=lax.Precision.HIGHEST)
    d_ref[...] = jnp.dot(h * adst_ref[...], s16,
                         preferred_element_type=jnp.float32, precision=lax.Precision.HIGHEST)


def _tables(x, W, asrc_vec, adst_vec):
    fin = x.shape[1]
    full = lambda s: pl.BlockSpec(s, lambda i: tuple(0 for _ in s))
    return pl.pallas_call(
        _tables_block,
        grid=(NTC,),
        in_specs=[pl.BlockSpec((BLK, fin), lambda i: (i, 0)),
                  full((fin, F)), full((1, F)), full((1, F))],
        out_specs=[pl.BlockSpec((BLK, GW), lambda i: (i, 0)),
                   pl.BlockSpec((BLK, DW), lambda i: (i, 0))],
        out_shape=[jax.ShapeDtypeStruct((NP, GW), jnp.float32),
                   jax.ShapeDtypeStruct((NP, DW), jnp.float32)],
    )(x, W, asrc_vec, adst_vec)


def _combine_block(a0_ref, a1_ref, b_ref, o_ref):
    t = a0_ref[...] + a1_ref[...]
    nmat = _sel((GW, F), lambda i, j: i == j)
    bmat = _sel((GW, F), lambda i, j: i == F + j // C)
    num = jnp.dot(t, nmat, preferred_element_type=jnp.float32, precision=lax.Precision.HIGHEST)
    den = jnp.dot(t, bmat, preferred_element_type=jnp.float32, precision=lax.Precision.HIGHEST)
    out = num / (den + 1e-16) + b_ref[...]
    o_ref[...] = jnp.where(out > 0.0, out, jnp.exp(out) - 1.0)


def _combine_elu(acc0, acc1, b):
    full = lambda s: pl.BlockSpec(s, lambda i: tuple(0 for _ in s))
    return pl.pallas_call(
        _combine_block,
        grid=(NTC,),
        in_specs=[pl.BlockSpec((BLK, GW), lambda i: (i, 0)),
                  pl.BlockSpec((BLK, GW), lambda i: (i, 0)),
                  full((1, F))],
        out_specs=pl.BlockSpec((BLK, F), lambda i: (i, 0)),
        out_shape=jax.ShapeDtypeStruct((NP, F), jnp.float32),
    )(acc0, acc1, b)


def _head_block(h_ref, wm1, bm1, g1, be1, wm2, bm2, g2, be2, wm3, bm3, g3,
                be3, wout, bout, o_ref):
    z = jnp.maximum(jnp.dot(h_ref[...], wm1[...],
                            preferred_element_type=jnp.float32, precision=lax.Precision.HIGHEST) + bm1[...], 0.0)
    z = g1[...] * z * _SQ + be1[...]
    z = jnp.maximum(jnp.dot(z, wm2[...],
                            preferred_element_type=jnp.float32, precision=lax.Precision.HIGHEST) + bm2[...], 0.0)
    z = g2[...] * z * _SQ + be2[...]
    z = jnp.maximum(jnp.dot(z, wm3[...],
                            preferred_element_type=jnp.float32, precision=lax.Precision.HIGHEST) + bm3[...], 0.0)
    z = g3[...] * z * _SQ + be3[...]
    o_ref[...] = jnp.dot(z, wout[...],
                         preferred_element_type=jnp.float32, precision=lax.Precision.HIGHEST) + bout[...]


def _mlp_head(h, Wm1, bm1, g1, be1, Wm2, bm2, g2, be2, Wm3, bm3, g3, be3,
              Wout, bout):
    full = lambda s: pl.BlockSpec(s, lambda i: tuple(0 for _ in s))
    return pl.pallas_call(
        _head_block,
        grid=(NTC,),
        in_specs=[pl.BlockSpec((BLK, F), lambda i: (i, 0)),
                  full((F, 32)), full((1, 32)), full((1, 32)), full((1, 32)),
                  full((32, 16)), full((1, 16)), full((1, 16)), full((1, 16)),
                  full((16, 8)), full((1, 8)), full((1, 8)), full((1, 8)),
                  full((8, 8)), full((1, 8))],
        out_specs=pl.BlockSpec((BLK, 8), lambda i: (i, 0)),
        out_shape=jax.ShapeDtypeStruct((NP, 8), jnp.float32),
    )(h, Wm1, bm1, g1, be1, Wm2, bm2, g2, be2, Wm3, bm3, g3, be3, Wout, bout)


def _edge_pass_body(g_hbm, d_hbm, src_hbm, dst_hbm, zero_hbm, out_hbm,
                    srcv, dstv, gbuf0, gbuf1, dbuf0, dbuf1, sbuf0, sbuf1,
                    acc, gsem0, gsem1, dsem0, dsem1):
    cid = lax.axis_index("c")
    sid = lax.axis_index("s")
    nb = src_hbm.shape[1]
    rpt = NP // NSUB

    pltpu.sync_copy(zero_hbm, acc.at[pl.ds(sid * rpt, rpt)])

    wid = sid * NCORE + cid
    pltpu.sync_copy(src_hbm.at[wid], srcv)
    pltpu.sync_copy(dst_hbm.at[wid], dstv)

    lane = lax.iota(jnp.int32, 16)
    hi = lane >> 3
    lo = lane & 7
    col_a = F + lo
    zero16 = jnp.zeros((16,), jnp.float32)
    bcast_cols = [F + 2 * j + hi for j in range(4)]

    @pl.loop(0, K)
    def _init(i):
        sbuf0[i, pl.ds(F, 16)] = zero16
        sbuf1[i, pl.ds(F, 16)] = zero16

    plsc.subcore_barrier()

    def compute(gb, db, sb):
        @pl.loop(0, K, step=2)
        def _pair(i):
            rowp = hi + i
            a = plsc.load_gather(gb, [rowp, col_a])
            d = plsc.load_gather(db, [rowp, lo])
            e = a + d
            e = jnp.maximum(e, e * 0.2)
            ex = jnp.exp(e)
            plsc.store_scatter(sb, [rowp, col_a], ex)
            for k in range(2):
                row = jnp.full((16,), i + k, dtype=jnp.int32)
                for j in range(4):
                    bc = plsc.load_gather(sb, [row, bcast_cols[j]])
                    sb[i + k, pl.ds(16 * j, 16)] = (
                        gb[i + k, pl.ds(16 * j, 16)] * bc)

    def fetch(b, gb, db, gs, ds):
        cg = pltpu.async_copy(g_hbm.at[srcv.at[b]], gb, gs)
        cd = pltpu.async_copy(d_hbm.at[dstv.at[b]], db, ds)
        return cg, cd

    def waitp(p):
        p[0].wait()
        p[1].wait()

    @pl.loop(0, nb)
    def _blk(b):
        waitp(fetch(b, gbuf0, dbuf0, gsem0, dsem0))
        compute(gbuf0, dbuf0, sbuf0)
        pltpu.sync_copy(sbuf0, acc.at[dstv.at[b]], add=True)

    plsc.subcore_barrier()
    pltpu.sync_copy(acc.at[pl.ds(sid * rpt, rpt)],
                    out_hbm.at[cid].at[pl.ds(sid * rpt, rpt)])


def _edge_pass(g, d, src3, dst3, zero_rows):
    mesh = plsc.VectorSubcoreMesh(core_axis_name="c", subcore_axis_name="s")
    nb = src3.shape[1]
    cp = pltpu.CompilerParams(needs_layout_passes=False,
                              use_tc_tiling_on_sc=False)
    kern = functools.partial(
        pl.kernel,
        compiler_params=cp,
        out_type=jax.ShapeDtypeStruct((NCORE, NP, GW), jnp.float32),
        mesh=mesh,
        scratch_types=[
            pltpu.VMEM((nb, K), jnp.int32),
            pltpu.VMEM((nb, K), jnp.int32),
            pltpu.VMEM((K, GW), jnp.float32),
            pltpu.VMEM((K, GW), jnp.float32),
            pltpu.VMEM((K, DW), jnp.float32),
            pltpu.VMEM((K, DW), jnp.float32),
            pltpu.VMEM((K, GW), jnp.float32),
            pltpu.VMEM((K, GW), jnp.float32),
            pltpu.VMEM_SHARED((NP, GW), jnp.float32),
            pltpu.SemaphoreType.DMA,
            pltpu.SemaphoreType.DMA,
            pltpu.SemaphoreType.DMA,
            pltpu.SemaphoreType.DMA,
        ],
    )(_edge_pass_body)
    return kern(g, d, src3, dst3, zero_rows)


def kernel(x, edge_index, W1, a_src1, a_dst1, b1, W2, a_src2, a_dst2, b2,
           Wm1, bm1, g1, be1, Wm2, bm2, g2, be2, Wm3, bm3, g3, be3,
           Wout, bout):
    n = x.shape[0]
    loop = jnp.arange(n, dtype=edge_index.dtype)
    src = jnp.concatenate([edge_index[0], loop])
    dst = jnp.concatenate([edge_index[1], loop])
    e_tot = src.shape[0]
    ept = ((e_tot + NWORK * K - 1) // (NWORK * K)) * K
    epad = NWORK * ept - e_tot
    trash = jnp.full((epad,), N, dtype=src.dtype)
    src3 = jnp.concatenate([src, trash]).reshape(NWORK, ept // K, K)
    dst3 = jnp.concatenate([dst, trash]).reshape(NWORK, ept // K, K)

    xp = jnp.pad(x, ((0, NP - n), (0, 0)))
    zero_rows = jnp.zeros((NP // NSUB, GW), jnp.float32)

    g1t, d1t = _tables(xp, W1, a_src1.reshape(1, F), a_dst1.reshape(1, F))
    acc1 = _edge_pass(g1t, d1t, src3, dst3, zero_rows)
    h1 = _combine_elu(acc1[0], acc1[1], b1.reshape(1, F))

    g2t, d2t = _tables(h1, W2, a_src2.reshape(1, F), a_dst2.reshape(1, F))
    acc2 = _edge_pass(g2t, d2t, src3, dst3, zero_rows)
    h2 = _combine_elu(acc2[0], acc2[1], b2.reshape(1, F))

    out = _mlp_head(h2, Wm1, bm1.reshape(1, 32), g1.reshape(1, 32),
                    be1.reshape(1, 32), Wm2, bm2.reshape(1, 16),
                    g2.reshape(1, 16), be2.reshape(1, 16), Wm3,
                    bm3.reshape(1, 8), g3.reshape(1, 8), be3.reshape(1, 8),
                    Wout, bout.reshape(1, 8))
    return out[:n]
